# Initial kernel scaffold; baseline (speedup 1.0000x reference)
import jax
import jax.numpy as jnp
from jax.experimental import pallas as pl


def kernel(x, edge_index, W1, b1, Wp, bp, W2, b2):
    raise NotImplementedError("write your pallas kernel here")


# speedup vs baseline: 1066.1486x; 1066.1486x over previous
import functools

import jax
import jax.numpy as jnp
from jax import lax
from jax.experimental import pallas as pl
from jax.experimental.pallas import tpu as pltpu
from jax.experimental.pallas import tpu_sc as plsc

C = 128
K = 32
NC = 2
NS = 16
NW = NC * NS
N_PER_W = 320
N_PAD = NW * N_PER_W
CH = 4
CHUNKS = N_PER_W // CH
NB = 512
CGRP = C // 16


def _stage1_body(x_ref, w1_ref, b1_ref, wp_ref, bp_ref, h_ref, g_ref):
    x = x_ref[...]
    h = lax.dot_general(x, w1_ref[...], (((1,), (1,)), ((), ())),
                        preferred_element_type=jnp.float32)
    h = jnp.maximum(h + b1_ref[...], 0.0)
    g = lax.dot_general(h, wp_ref[...], (((1,), (1,)), ((), ())),
                        preferred_element_type=jnp.float32)
    g = jnp.maximum(g + bp_ref[...], 0.0)
    h_ref[...] = h
    g_ref[...] = g


def _stage1(xt, W1, b1, Wp, bp):
    grid = (N_PAD // NB,)
    return pl.pallas_call(
        _stage1_body,
        grid=grid,
        in_specs=[
            pl.BlockSpec((NB, C), lambda i: (i, 0)),
            pl.BlockSpec((C, C), lambda i: (0, 0)),
            pl.BlockSpec((1, C), lambda i: (0, 0)),
            pl.BlockSpec((C, C), lambda i: (0, 0)),
            pl.BlockSpec((1, C), lambda i: (0, 0)),
        ],
        out_specs=[
            pl.BlockSpec((NB, C), lambda i: (i, 0)),
            pl.BlockSpec((NB, C), lambda i: (i, 0)),
        ],
        out_shape=[
            jax.ShapeDtypeStruct((N_PAD, C), jnp.float32),
            jax.ShapeDtypeStruct((N_PAD, C), jnp.float32),
        ],
    )(xt, W1, b1, Wp, bp)


def _sc_pool_body(g_hbm, idx_hbm, out_hbm, idx_v, rows_v, out_v, sem):
    cid = lax.axis_index("c")
    sid = lax.axis_index("s")
    wid = sid * NC + cid
    base_node = wid * N_PER_W

    def chunk_body(ci, carry):
        node0 = base_node + ci * CH
        pltpu.sync_copy(idx_hbm.at[pl.ds(node0 * K, CH * K)], idx_v)
        pltpu.async_copy(g_hbm.at[idx_v], rows_v, sem).wait()
        for j in range(CH):
            accs = [rows_v[j * K, pl.ds(c * 16, 16)] for c in range(CGRP)]
            for k in range(1, K):
                for c in range(CGRP):
                    accs[c] = jnp.maximum(
                        accs[c], rows_v[j * K + k, pl.ds(c * 16, 16)])
            for c in range(CGRP):
                out_v[j, pl.ds(c * 16, 16)] = accs[c]
        pltpu.sync_copy(out_v, out_hbm.at[pl.ds(node0, CH)])
        return carry

    lax.fori_loop(0, CHUNKS, chunk_body, 0)


_sc_pool = functools.partial(
    pl.kernel,
    out_type=jax.ShapeDtypeStruct((N_PAD, C), jnp.float32),
    mesh=plsc.VectorSubcoreMesh(core_axis_name="c", subcore_axis_name="s"),
    scratch_types=[
        pltpu.VMEM((CH * K,), jnp.int32),
        pltpu.VMEM((CH * K, C), jnp.float32),
        pltpu.VMEM((CH, C), jnp.float32),
        pltpu.SemaphoreType.DMA,
    ],
)(_sc_pool_body)


def _stage3_body(h_ref, p_ref, w2a_ref, w2b_ref, b2_ref, o_ref):
    o = lax.dot_general(w2a_ref[...], h_ref[...], (((1,), (1,)), ((), ())),
                        preferred_element_type=jnp.float32)
    o = o + lax.dot_general(w2b_ref[...], p_ref[...], (((1,), (1,)), ((), ())),
                            preferred_element_type=jnp.float32)
    o_ref[...] = jnp.maximum(o + b2_ref[...], 0.0)


def _stage3(h, pooled, W2a, W2b, b2):
    grid = (N_PAD // NB,)
    return pl.pallas_call(
        _stage3_body,
        grid=grid,
        in_specs=[
            pl.BlockSpec((NB, C), lambda i: (i, 0)),
            pl.BlockSpec((NB, C), lambda i: (i, 0)),
            pl.BlockSpec((C, C), lambda i: (0, 0)),
            pl.BlockSpec((C, C), lambda i: (0, 0)),
            pl.BlockSpec((C, 1), lambda i: (0, 0)),
        ],
        out_specs=pl.BlockSpec((C, NB), lambda i: (0, i)),
        out_shape=jax.ShapeDtypeStruct((C, N_PAD), jnp.float32),
    )(h, pooled, W2a, W2b, b2)


def kernel(x, edge_index, W1, b1, Wp, bp, W2, b2):
    n = x.shape[2]
    xt = jnp.pad(x[0, :, :, 0].T, ((0, N_PAD - n), (0, 0)))
    idx = edge_index[0, 0]
    idx_flat = jnp.pad(idx, ((0, N_PAD - n), (0, 0))).reshape(-1)
    h, g = _stage1(xt, W1, b1.reshape(1, C), Wp, bp.reshape(1, C))
    pooled = _sc_pool(g, idx_flat)
    out = _stage3(h, pooled, W2[:, :C], W2[:, C:], b2.reshape(C, 1))
    return out[:, :n].reshape(1, C, n, 1)


# speedup vs baseline: 1521.4345x; 1.4270x over previous
import functools

import jax
import jax.numpy as jnp
from jax import lax
from jax.experimental import pallas as pl
from jax.experimental.pallas import tpu as pltpu
from jax.experimental.pallas import tpu_sc as plsc

C = 128
K = 32
NC = 2
NS = 16
NW = NC * NS
N_PER_W = 320
N_PAD = NW * N_PER_W
CH = 4
CHUNKS = N_PER_W // CH
NB = 512
CGRP = C // 16


def _stage1_body(x_ref, w1_ref, b1_ref, wp_ref, bp_ref, h_ref, g_ref):
    x = x_ref[...]
    h = lax.dot_general(x, w1_ref[...], (((1,), (1,)), ((), ())),
                        preferred_element_type=jnp.float32)
    h = jnp.maximum(h + b1_ref[...], 0.0)
    g = lax.dot_general(h, wp_ref[...], (((1,), (1,)), ((), ())),
                        preferred_element_type=jnp.float32)
    g = jnp.maximum(g + bp_ref[...], 0.0)
    h_ref[...] = h
    g_ref[...] = g


def _stage1(xt, W1, b1, Wp, bp):
    grid = (N_PAD // NB,)
    return pl.pallas_call(
        _stage1_body,
        grid=grid,
        in_specs=[
            pl.BlockSpec((NB, C), lambda i: (i, 0)),
            pl.BlockSpec((C, C), lambda i: (0, 0)),
            pl.BlockSpec((1, C), lambda i: (0, 0)),
            pl.BlockSpec((C, C), lambda i: (0, 0)),
            pl.BlockSpec((1, C), lambda i: (0, 0)),
        ],
        out_specs=[
            pl.BlockSpec((NB, C), lambda i: (i, 0)),
            pl.BlockSpec((NB, C), lambda i: (i, 0)),
        ],
        out_shape=[
            jax.ShapeDtypeStruct((N_PAD, C), jnp.float32),
            jax.ShapeDtypeStruct((N_PAD, C), jnp.float32),
        ],
    )(xt, W1, b1, Wp, bp)


PAIRS = CHUNKS // 2


def _sc_pool_body(g_hbm, idx_hbm, out_hbm, idx_v, rows0, rows1, out_v,
                  sem0, sem1):
    cid = lax.axis_index("c")
    sid = lax.axis_index("s")
    wid = sid * NC + cid
    base_node = wid * N_PER_W

    pltpu.sync_copy(idx_hbm.at[pl.ds(base_node * K, N_PER_W * K)], idx_v)

    def issue(ci, rows_v, sem):
        pltpu.async_copy(
            g_hbm.at[idx_v.at[pl.ds(ci * (CH * K), CH * K)]], rows_v, sem)

    def drain(rows_v, sem):
        pltpu.make_async_copy(
            g_hbm.at[idx_v.at[pl.ds(0, CH * K)]], rows_v, sem).wait()

    def compute(rows_v, ci):
        for j in range(CH):
            accs = [rows_v[j * K, pl.ds(c * 16, 16)] for c in range(CGRP)]
            for k in range(1, K):
                for c in range(CGRP):
                    accs[c] = jnp.maximum(
                        accs[c], rows_v[j * K + k, pl.ds(c * 16, 16)])
            for c in range(CGRP):
                out_v[ci * CH + j, pl.ds(c * 16, 16)] = accs[c]

    issue(0, rows0, sem0)

    def pair_body(p, carry):
        c0 = 2 * p
        issue(c0 + 1, rows1, sem1)
        drain(rows0, sem0)
        compute(rows0, c0)

        @pl.when(p < PAIRS - 1)
        def _():
            issue(c0 + 2, rows0, sem0)

        drain(rows1, sem1)
        compute(rows1, c0 + 1)
        return carry

    lax.fori_loop(0, PAIRS, pair_body, 0)
    pltpu.sync_copy(out_v, out_hbm.at[pl.ds(base_node, N_PER_W)])


_sc_pool = functools.partial(
    pl.kernel,
    out_type=jax.ShapeDtypeStruct((N_PAD, C), jnp.float32),
    mesh=plsc.VectorSubcoreMesh(core_axis_name="c", subcore_axis_name="s"),
    scratch_types=[
        pltpu.VMEM((N_PER_W * K,), jnp.int32),
        pltpu.VMEM((CH * K, C), jnp.float32),
        pltpu.VMEM((CH * K, C), jnp.float32),
        pltpu.VMEM((N_PER_W, C), jnp.float32),
        pltpu.SemaphoreType.DMA,
        pltpu.SemaphoreType.DMA,
    ],
)(_sc_pool_body)


def _stage3_body(h_ref, p_ref, w2a_ref, w2b_ref, b2_ref, o_ref):
    o = lax.dot_general(w2a_ref[...], h_ref[...], (((1,), (1,)), ((), ())),
                        preferred_element_type=jnp.float32)
    o = o + lax.dot_general(w2b_ref[...], p_ref[...], (((1,), (1,)), ((), ())),
                            preferred_element_type=jnp.float32)
    o_ref[...] = jnp.maximum(o + b2_ref[...], 0.0)


def _stage3(h, pooled, W2a, W2b, b2):
    grid = (N_PAD // NB,)
    return pl.pallas_call(
        _stage3_body,
        grid=grid,
        in_specs=[
            pl.BlockSpec((NB, C), lambda i: (i, 0)),
            pl.BlockSpec((NB, C), lambda i: (i, 0)),
            pl.BlockSpec((C, C), lambda i: (0, 0)),
            pl.BlockSpec((C, C), lambda i: (0, 0)),
            pl.BlockSpec((C, 1), lambda i: (0, 0)),
        ],
        out_specs=pl.BlockSpec((C, NB), lambda i: (0, i)),
        out_shape=jax.ShapeDtypeStruct((C, N_PAD), jnp.float32),
    )(h, pooled, W2a, W2b, b2)


def kernel(x, edge_index, W1, b1, Wp, bp, W2, b2):
    n = x.shape[2]
    xt = jnp.pad(x[0, :, :, 0].T, ((0, N_PAD - n), (0, 0)))
    idx = edge_index[0, 0]
    idx_flat = jnp.pad(idx, ((0, N_PAD - n), (0, 0))).reshape(-1)
    h, g = _stage1(xt, W1, b1.reshape(1, C), Wp, bp.reshape(1, C))
    pooled = _sc_pool(g, idx_flat)
    out = _stage3(h, pooled, W2[:, :C], W2[:, C:], b2.reshape(C, 1))
    return out[:, :n].reshape(1, C, n, 1)


# speedup vs baseline: 1889.3778x; 1.2418x over previous
import functools

import jax
import jax.numpy as jnp
from jax import lax
from jax.experimental import pallas as pl
from jax.experimental.pallas import tpu as pltpu
from jax.experimental.pallas import tpu_sc as plsc

C = 128
CP = C // 2
K = 32
NC = 2
NS = 16
NW = NC * NS
N_PER_W = 320
N_PAD = NW * N_PER_W
CH = 4
CHUNKS = N_PER_W // CH
NBUF = 4
GROUPS = CHUNKS // NBUF
NB = 512


def _stage1_body(x_ref, w1_ref, b1_ref, wp_ref, bp_ref, h_ref, g_ref):
    x = x_ref[...]
    h = lax.dot_general(x, w1_ref[...], (((1,), (1,)), ((), ())),
                        preferred_element_type=jnp.float32)
    h = jnp.maximum(h + b1_ref[...], 0.0)
    g = lax.dot_general(h, wp_ref[...], (((1,), (1,)), ((), ())),
                        preferred_element_type=jnp.float32)
    g = jnp.maximum(g + bp_ref[...], 0.0)
    h_ref[...] = h
    g_ref[...] = g.astype(jnp.bfloat16)


def _stage1(xt, W1, b1, Wp, bp):
    grid = (N_PAD // NB,)
    return pl.pallas_call(
        _stage1_body,
        grid=grid,
        in_specs=[
            pl.BlockSpec((NB, C), lambda i: (i, 0)),
            pl.BlockSpec((C, C), lambda i: (0, 0)),
            pl.BlockSpec((1, C), lambda i: (0, 0)),
            pl.BlockSpec((C, C), lambda i: (0, 0)),
            pl.BlockSpec((1, C), lambda i: (0, 0)),
        ],
        out_specs=[
            pl.BlockSpec((NB, C), lambda i: (i, 0)),
            pl.BlockSpec((NB, C), lambda i: (i, 0)),
        ],
        out_shape=[
            jax.ShapeDtypeStruct((N_PAD, C), jnp.float32),
            jax.ShapeDtypeStruct((N_PAD, C), jnp.bfloat16),
        ],
    )(xt, W1, b1, Wp, bp)


_MASK = 0xFFFF


def _sc_pool_body(g_hbm, idx_hbm, out_hbm, idx_v, rows, out_v, sems):
    cid = lax.axis_index("c")
    sid = lax.axis_index("s")
    wid = sid * NC + cid
    base_node = wid * N_PER_W

    pltpu.sync_copy(idx_hbm.at[pl.ds(base_node * K, N_PER_W * K)], idx_v)

    def issue(ci, b):
        pltpu.async_copy(
            g_hbm.at[idx_v.at[pl.ds(ci * (CH * K), CH * K)]], rows[b],
            sems[b])

    def drain(b):
        pltpu.make_async_copy(
            g_hbm.at[idx_v.at[pl.ds(0, CH * K)]], rows[b], sems[b]).wait()

    def compute(b, ci):
        rows_v = rows[b]

        def j_body(j, carry):
            row0 = j * K
            for c in range(CP // 16):
                acc_lo = None
                acc_hi = None
                for k0 in range(0, K, 8):
                    los, his = [], []
                    for k in range(k0, k0 + 8):
                        xv = rows_v[row0 + k, pl.ds(c * 16, 16)]
                        los.append(xv & _MASK)
                        his.append(lax.shift_right_logical(xv, 16))
                    while len(los) > 1:
                        los = [jnp.maximum(los[2 * i], los[2 * i + 1])
                               for i in range(len(los) // 2)]
                        his = [jnp.maximum(his[2 * i], his[2 * i + 1])
                               for i in range(len(his) // 2)]
                    if acc_lo is None:
                        acc_lo, acc_hi = los[0], his[0]
                    else:
                        acc_lo = jnp.maximum(acc_lo, los[0])
                        acc_hi = jnp.maximum(acc_hi, his[0])
                out_v[ci * CH + j, pl.ds(c * 16, 16)] = (
                    lax.shift_left(acc_hi, 16) | acc_lo)
            return carry

        lax.fori_loop(0, CH, j_body, 0)

    for b in range(NBUF - 1):
        issue(b, b)

    def group_body(gi, carry):
        c0 = gi * NBUF
        for b in range(NBUF):
            ci = c0 + b

            @pl.when(ci + NBUF - 1 < CHUNKS)
            def _():
                issue(ci + NBUF - 1, (b + NBUF - 1) % NBUF)

            drain(b)
            compute(b, ci)
        return carry

    lax.fori_loop(0, GROUPS, group_body, 0)
    pltpu.sync_copy(out_v, out_hbm.at[pl.ds(base_node, N_PER_W)])


_sc_pool = functools.partial(
    pl.kernel,
    out_type=jax.ShapeDtypeStruct((N_PAD, CP), jnp.int32),
    mesh=plsc.VectorSubcoreMesh(core_axis_name="c", subcore_axis_name="s"),
    scratch_types=[
        pltpu.VMEM((N_PER_W * K,), jnp.int32),
        [pltpu.VMEM((CH * K, CP), jnp.int32) for _ in range(NBUF)],
        pltpu.VMEM((N_PER_W, CP), jnp.int32),
        [pltpu.SemaphoreType.DMA for _ in range(NBUF)],
    ],
    compiler_params=pltpu.CompilerParams(use_tc_tiling_on_sc=False),
)(_sc_pool_body)


def _stage3_body(h_ref, p_ref, w2a_ref, w2b_ref, b2_ref, o_ref):
    o = lax.dot_general(w2a_ref[...], h_ref[...], (((1,), (1,)), ((), ())),
                        preferred_element_type=jnp.float32)
    p = p_ref[...].astype(jnp.float32)
    o = o + lax.dot_general(w2b_ref[...], p, (((1,), (1,)), ((), ())),
                            preferred_element_type=jnp.float32)
    o_ref[...] = jnp.maximum(o + b2_ref[...], 0.0)


def _stage3(h, pooled, W2a, W2b, b2):
    grid = (N_PAD // NB,)
    return pl.pallas_call(
        _stage3_body,
        grid=grid,
        in_specs=[
            pl.BlockSpec((NB, C), lambda i: (i, 0)),
            pl.BlockSpec((NB, C), lambda i: (i, 0)),
            pl.BlockSpec((C, C), lambda i: (0, 0)),
            pl.BlockSpec((C, C), lambda i: (0, 0)),
            pl.BlockSpec((C, 1), lambda i: (0, 0)),
        ],
        out_specs=pl.BlockSpec((C, NB), lambda i: (0, i)),
        out_shape=jax.ShapeDtypeStruct((C, N_PAD), jnp.float32),
    )(h, pooled, W2a, W2b, b2)


def kernel(x, edge_index, W1, b1, Wp, bp, W2, b2):
    n = x.shape[2]
    xt = jnp.pad(x[0, :, :, 0].T, ((0, N_PAD - n), (0, 0)))
    idx = edge_index[0, 0]
    idx_flat = jnp.pad(idx, ((0, N_PAD - n), (0, 0))).reshape(-1)
    h, g = _stage1(xt, W1, b1.reshape(1, C), Wp, bp.reshape(1, C))
    g_packed = lax.bitcast_convert_type(g.reshape(N_PAD, CP, 2), jnp.int32)
    pooled_packed = _sc_pool(g_packed, idx_flat)
    pooled = lax.bitcast_convert_type(
        pooled_packed, jnp.bfloat16).reshape(N_PAD, C)
    out = _stage3(h, pooled, W2[:, :C], W2[:, C:], b2.reshape(C, 1))
    return out[:, :n].reshape(1, C, n, 1)


# speedup vs baseline: 3839.5505x; 2.0322x over previous
import functools

import jax
import jax.numpy as jnp
from jax import lax
from jax.experimental import pallas as pl
from jax.experimental.pallas import tpu as pltpu
from jax.experimental.pallas import tpu_sc as plsc

C = 128
CP = C // 2
K = 32
NC = 2
NS = 16
NW = NC * NS
N_PER_W = 320
N_PAD = NW * N_PER_W
CH = 4
CHUNKS = N_PER_W // CH
NBUF = 4
GROUPS = CHUNKS // NBUF
NB = 512


def _stage1_body(x_ref, w1_ref, b1_ref, wp_ref, bp_ref, h_ref, g_ref):
    x = x_ref[...]
    h = lax.dot_general(x, w1_ref[...], (((1,), (1,)), ((), ())),
                        preferred_element_type=jnp.float32)
    h = jnp.maximum(h + b1_ref[...], 0.0)
    g = lax.dot_general(h, wp_ref[...], (((1,), (1,)), ((), ())),
                        preferred_element_type=jnp.float32)
    g = jnp.maximum(g + bp_ref[...], 0.0)
    h_ref[...] = h
    g_ref[...] = g.astype(jnp.bfloat16)


def _stage1(xt, W1, b1, Wp, bp):
    grid = (N_PAD // NB,)
    return pl.pallas_call(
        _stage1_body,
        grid=grid,
        in_specs=[
            pl.BlockSpec((NB, C), lambda i: (i, 0)),
            pl.BlockSpec((C, C), lambda i: (0, 0)),
            pl.BlockSpec((1, C), lambda i: (0, 0)),
            pl.BlockSpec((C, C), lambda i: (0, 0)),
            pl.BlockSpec((1, C), lambda i: (0, 0)),
        ],
        out_specs=[
            pl.BlockSpec((NB, C), lambda i: (i, 0)),
            pl.BlockSpec((NB, C), lambda i: (i, 0)),
        ],
        out_shape=[
            jax.ShapeDtypeStruct((N_PAD, C), jnp.float32),
            jax.ShapeDtypeStruct((N_PAD, C), jnp.bfloat16),
        ],
    )(xt, W1, b1, Wp, bp)


_MASK = 0xFFFF


def _sc_pool_body(g_hbm, idx_hbm, out_hbm, idx_v, rows, out_v, g_sh, sems):
    cid = lax.axis_index("c")
    sid = lax.axis_index("s")
    wid = sid * NC + cid
    base_node = wid * N_PER_W

    rows_per_tile = N_PAD // NS
    pltpu.sync_copy(g_hbm.at[pl.ds(sid * rows_per_tile, rows_per_tile)],
                    g_sh.at[pl.ds(sid * rows_per_tile, rows_per_tile)])
    pltpu.sync_copy(idx_hbm.at[pl.ds(base_node * K, N_PER_W * K)], idx_v)
    plsc.subcore_barrier()

    def issue(ci, b):
        pltpu.async_copy(
            g_sh.at[idx_v.at[pl.ds(ci * (CH * K), CH * K)]], rows[b],
            sems[b])

    def drain(b):
        pltpu.make_async_copy(
            g_sh.at[idx_v.at[pl.ds(0, CH * K)]], rows[b], sems[b]).wait()

    def compute(b, ci):
        rows_v = rows[b]

        def j_body(j, carry):
            row0 = j * K
            for c in range(CP // 16):
                acc_lo = None
                acc_hi = None
                for k0 in range(0, K, 8):
                    los, his = [], []
                    for k in range(k0, k0 + 8):
                        xv = rows_v[row0 + k, pl.ds(c * 16, 16)]
                        los.append(xv & _MASK)
                        his.append(lax.shift_right_logical(xv, 16))
                    while len(los) > 1:
                        los = [jnp.maximum(los[2 * i], los[2 * i + 1])
                               for i in range(len(los) // 2)]
                        his = [jnp.maximum(his[2 * i], his[2 * i + 1])
                               for i in range(len(his) // 2)]
                    if acc_lo is None:
                        acc_lo, acc_hi = los[0], his[0]
                    else:
                        acc_lo = jnp.maximum(acc_lo, los[0])
                        acc_hi = jnp.maximum(acc_hi, his[0])
                out_v[ci * CH + j, pl.ds(c * 16, 16)] = (
                    lax.shift_left(acc_hi, 16) | acc_lo)
            return carry

        lax.fori_loop(0, CH, j_body, 0)

    for b in range(NBUF - 1):
        issue(b, b)

    def group_body(gi, carry):
        c0 = gi * NBUF
        for b in range(NBUF):
            ci = c0 + b

            @pl.when(ci + NBUF - 1 < CHUNKS)
            def _():
                issue(ci + NBUF - 1, (b + NBUF - 1) % NBUF)

            drain(b)
            compute(b, ci)
        return carry

    lax.fori_loop(0, GROUPS, group_body, 0)
    pltpu.sync_copy(out_v, out_hbm.at[pl.ds(base_node, N_PER_W)])


_sc_pool = functools.partial(
    pl.kernel,
    out_type=jax.ShapeDtypeStruct((N_PAD, CP), jnp.int32),
    mesh=plsc.VectorSubcoreMesh(core_axis_name="c", subcore_axis_name="s"),
    scratch_types=[
        pltpu.VMEM((N_PER_W * K,), jnp.int32),
        [pltpu.VMEM((CH * K, CP), jnp.int32) for _ in range(NBUF)],
        pltpu.VMEM((N_PER_W, CP), jnp.int32),
        pltpu.VMEM_SHARED((N_PAD, CP), jnp.int32),
        [pltpu.SemaphoreType.DMA for _ in range(NBUF)],
    ],
    compiler_params=pltpu.CompilerParams(use_tc_tiling_on_sc=False),
)(_sc_pool_body)


def _stage3_body(h_ref, p_ref, w2a_ref, w2b_ref, b2_ref, o_ref):
    o = lax.dot_general(w2a_ref[...], h_ref[...], (((1,), (1,)), ((), ())),
                        preferred_element_type=jnp.float32)
    p = p_ref[...].astype(jnp.float32)
    o = o + lax.dot_general(w2b_ref[...], p, (((1,), (1,)), ((), ())),
                            preferred_element_type=jnp.float32)
    o_ref[...] = jnp.maximum(o + b2_ref[...], 0.0)


def _stage3(h, pooled, W2a, W2b, b2):
    grid = (N_PAD // NB,)
    return pl.pallas_call(
        _stage3_body,
        grid=grid,
        in_specs=[
            pl.BlockSpec((NB, C), lambda i: (i, 0)),
            pl.BlockSpec((NB, C), lambda i: (i, 0)),
            pl.BlockSpec((C, C), lambda i: (0, 0)),
            pl.BlockSpec((C, C), lambda i: (0, 0)),
            pl.BlockSpec((C, 1), lambda i: (0, 0)),
        ],
        out_specs=pl.BlockSpec((C, NB), lambda i: (0, i)),
        out_shape=jax.ShapeDtypeStruct((C, N_PAD), jnp.float32),
    )(h, pooled, W2a, W2b, b2)


def kernel(x, edge_index, W1, b1, Wp, bp, W2, b2):
    n = x.shape[2]
    xt = jnp.pad(x[0, :, :, 0].T, ((0, N_PAD - n), (0, 0)))
    idx = edge_index[0, 0]
    idx_flat = jnp.pad(idx, ((0, N_PAD - n), (0, 0))).reshape(-1)
    h, g = _stage1(xt, W1, b1.reshape(1, C), Wp, bp.reshape(1, C))
    g_packed = lax.bitcast_convert_type(g.reshape(N_PAD, CP, 2), jnp.int32)
    pooled_packed = _sc_pool(g_packed, idx_flat)
    pooled = lax.bitcast_convert_type(
        pooled_packed, jnp.bfloat16).reshape(N_PAD, C)
    out = _stage3(h, pooled, W2[:, :C], W2[:, C:], b2.reshape(C, 1))
    return out[:, :n].reshape(1, C, n, 1)


# speedup vs baseline: 5469.9967x; 1.4246x over previous
import functools

import jax
import jax.numpy as jnp
from jax import lax
from jax.experimental import pallas as pl
from jax.experimental.pallas import tpu as pltpu
from jax.experimental.pallas import tpu_sc as plsc

C = 128
CP = C // 2
K = 32
NC = 2
NS = 16
NW = NC * NS
N_PER_W = 320
N_PAD = NW * N_PER_W
CH = 4
CHUNKS = N_PER_W // CH
NBUF = 4
GROUPS = CHUNKS // NBUF
NB = 512


def _stage1_body(x_ref, w1_ref, b1_ref, wp_ref, bp_ref, h_ref, g_ref):
    x = x_ref[...]
    h = lax.dot_general(x, w1_ref[...], (((1,), (1,)), ((), ())),
                        preferred_element_type=jnp.float32)
    h = jnp.maximum(h + b1_ref[...], 0.0)
    g = lax.dot_general(h, wp_ref[...], (((1,), (1,)), ((), ())),
                        preferred_element_type=jnp.float32)
    g = jnp.maximum(g + bp_ref[...], 0.0)
    h_ref[...] = h
    lo = lax.bitcast_convert_type(
        g[:, :CP].astype(jnp.bfloat16), jnp.uint16).astype(jnp.int32)
    hi = lax.bitcast_convert_type(
        g[:, CP:].astype(jnp.bfloat16), jnp.uint16).astype(jnp.int32)
    g_ref[...] = lo | lax.shift_left(hi, 16)


def _stage1(xt, W1, b1, Wp, bp):
    grid = (N_PAD // NB,)
    return pl.pallas_call(
        _stage1_body,
        grid=grid,
        in_specs=[
            pl.BlockSpec((NB, C), lambda i: (i, 0)),
            pl.BlockSpec((C, C), lambda i: (0, 0)),
            pl.BlockSpec((1, C), lambda i: (0, 0)),
            pl.BlockSpec((C, C), lambda i: (0, 0)),
            pl.BlockSpec((1, C), lambda i: (0, 0)),
        ],
        out_specs=[
            pl.BlockSpec((NB, C), lambda i: (i, 0)),
            pl.BlockSpec((NB, CP), lambda i: (i, 0)),
        ],
        out_shape=[
            jax.ShapeDtypeStruct((N_PAD, C), jnp.float32),
            jax.ShapeDtypeStruct((N_PAD, CP), jnp.int32),
        ],
    )(xt, W1, b1, Wp, bp)


_MASK = 0xFFFF


def _sc_pool_body(g_hbm, idx_hbm, out_hbm, idx_v, rows, out_v, g_sh, sems):
    cid = lax.axis_index("c")
    sid = lax.axis_index("s")
    wid = sid * NC + cid
    base_node = wid * N_PER_W

    rows_per_tile = N_PAD // NS
    pltpu.sync_copy(g_hbm.at[pl.ds(sid * rows_per_tile, rows_per_tile)],
                    g_sh.at[pl.ds(sid * rows_per_tile, rows_per_tile)])
    pltpu.sync_copy(idx_hbm.at[pl.ds(base_node * K, N_PER_W * K)], idx_v)
    plsc.subcore_barrier()

    def issue(ci, b):
        pltpu.async_copy(
            g_sh.at[idx_v.at[pl.ds(ci * (CH * K), CH * K)]], rows[b],
            sems[b])

    def drain(b):
        pltpu.make_async_copy(
            g_sh.at[idx_v.at[pl.ds(0, CH * K)]], rows[b], sems[b]).wait()

    def compute(b, ci):
        rows_v = rows[b]

        def j_body(j, carry):
            row0 = j * K
            for c in range(CP // 16):
                acc_lo = None
                acc_hi = None
                for k0 in range(0, K, 8):
                    los, his = [], []
                    for k in range(k0, k0 + 8):
                        xv = rows_v[row0 + k, pl.ds(c * 16, 16)]
                        los.append(xv & _MASK)
                        his.append(lax.shift_right_logical(xv, 16))
                    while len(los) > 1:
                        los = [jnp.maximum(los[2 * i], los[2 * i + 1])
                               for i in range(len(los) // 2)]
                        his = [jnp.maximum(his[2 * i], his[2 * i + 1])
                               for i in range(len(his) // 2)]
                    if acc_lo is None:
                        acc_lo, acc_hi = los[0], his[0]
                    else:
                        acc_lo = jnp.maximum(acc_lo, los[0])
                        acc_hi = jnp.maximum(acc_hi, his[0])
                out_v[ci * CH + j, pl.ds(c * 16, 16)] = (
                    lax.shift_left(acc_hi, 16) | acc_lo)
            return carry

        lax.fori_loop(0, CH, j_body, 0)

    for b in range(NBUF - 1):
        issue(b, b)

    def group_body(gi, carry):
        c0 = gi * NBUF
        for b in range(NBUF):
            ci = c0 + b

            @pl.when(ci + NBUF - 1 < CHUNKS)
            def _():
                issue(ci + NBUF - 1, (b + NBUF - 1) % NBUF)

            drain(b)
            compute(b, ci)
        return carry

    lax.fori_loop(0, GROUPS, group_body, 0)
    pltpu.sync_copy(out_v, out_hbm.at[pl.ds(base_node, N_PER_W)])


_sc_pool = functools.partial(
    pl.kernel,
    out_type=jax.ShapeDtypeStruct((N_PAD, CP), jnp.int32),
    mesh=plsc.VectorSubcoreMesh(core_axis_name="c", subcore_axis_name="s"),
    scratch_types=[
        pltpu.VMEM((N_PER_W * K,), jnp.int32),
        [pltpu.VMEM((CH * K, CP), jnp.int32) for _ in range(NBUF)],
        pltpu.VMEM((N_PER_W, CP), jnp.int32),
        pltpu.VMEM_SHARED((N_PAD, CP), jnp.int32),
        [pltpu.SemaphoreType.DMA for _ in range(NBUF)],
    ],
    compiler_params=pltpu.CompilerParams(use_tc_tiling_on_sc=False),
)(_sc_pool_body)


def _stage3_body(h_ref, p_ref, w2a_ref, w2blo_ref, w2bhi_ref, b2_ref, o_ref):
    o = lax.dot_general(w2a_ref[...], h_ref[...], (((1,), (1,)), ((), ())),
                        preferred_element_type=jnp.float32)
    p = p_ref[...]
    p_lo = lax.bitcast_convert_type(lax.shift_left(p, 16), jnp.float32)
    p_hi = lax.bitcast_convert_type(p & (-65536), jnp.float32)
    o = o + lax.dot_general(w2blo_ref[...], p_lo, (((1,), (1,)), ((), ())),
                            preferred_element_type=jnp.float32)
    o = o + lax.dot_general(w2bhi_ref[...], p_hi, (((1,), (1,)), ((), ())),
                            preferred_element_type=jnp.float32)
    o_ref[...] = jnp.maximum(o + b2_ref[...], 0.0)


def _stage3(h, pooled, W2a, W2blo, W2bhi, b2, n):
    return pl.pallas_call(
        _stage3_body,
        grid=(1,),
        in_specs=[
            pl.BlockSpec((n, C), lambda i: (0, 0)),
            pl.BlockSpec((n, CP), lambda i: (0, 0)),
            pl.BlockSpec((C, C), lambda i: (0, 0)),
            pl.BlockSpec((C, CP), lambda i: (0, 0)),
            pl.BlockSpec((C, CP), lambda i: (0, 0)),
            pl.BlockSpec((C, 1), lambda i: (0, 0)),
        ],
        out_specs=pl.BlockSpec((C, n), lambda i: (0, 0)),
        out_shape=jax.ShapeDtypeStruct((C, n), jnp.float32),
    )(h, pooled, W2a, W2blo, W2bhi, b2)


def kernel(x, edge_index, W1, b1, Wp, bp, W2, b2):
    n = x.shape[2]
    xt = jnp.pad(x[0, :, :, 0].T, ((0, N_PAD - n), (0, 0)))
    idx = edge_index[0, 0]
    idx_flat = jnp.pad(idx, ((0, N_PAD - n), (0, 0))).reshape(-1)
    h, g_packed = _stage1(xt, W1, b1.reshape(1, C), Wp, bp.reshape(1, C))
    pooled_packed = _sc_pool(g_packed, idx_flat)
    out = _stage3(h, pooled_packed, W2[:, :C], W2[:, C:C + CP],
                  W2[:, C + CP:], b2.reshape(C, 1), n)
    return out.reshape(1, C, n, 1)


# speedup vs baseline: 5526.9902x; 1.0104x over previous
import functools

import jax
import jax.numpy as jnp
from jax import lax
from jax.experimental import pallas as pl
from jax.experimental.pallas import tpu as pltpu
from jax.experimental.pallas import tpu_sc as plsc

C = 128
CP = C // 2
K = 32
NC = 2
NS = 16
NW = NC * NS
N_PER_W = 320
N_PAD = NW * N_PER_W
N_REAL = 10000
CH = 4
CHUNKS = N_PER_W // CH
NBUF = 4
GROUPS = CHUNKS // NBUF
NB = 512


def _stage1_body(x_ref, w1_ref, b1_ref, wp_ref, bp_ref, h_ref, g_ref):
    x = x_ref[...]
    h = lax.dot_general(x, w1_ref[...], (((0,), (1,)), ((), ())),
                        preferred_element_type=jnp.float32)
    h = jnp.maximum(h + b1_ref[...], 0.0)
    g = lax.dot_general(h, wp_ref[...], (((1,), (1,)), ((), ())),
                        preferred_element_type=jnp.float32)
    g = jnp.maximum(g + bp_ref[...], 0.0)
    h_ref[...] = h
    lo = lax.bitcast_convert_type(
        g[:, :CP].astype(jnp.bfloat16), jnp.uint16).astype(jnp.int32)
    hi = lax.bitcast_convert_type(
        g[:, CP:].astype(jnp.bfloat16), jnp.uint16).astype(jnp.int32)
    g_ref[...] = lo | lax.shift_left(hi, 16)


def _stage1(x2, W1, b1, Wp, bp, n):
    return pl.pallas_call(
        _stage1_body,
        grid=(1,),
        in_specs=[
            pl.BlockSpec((C, n), lambda i: (0, 0)),
            pl.BlockSpec((C, C), lambda i: (0, 0)),
            pl.BlockSpec((1, C), lambda i: (0, 0)),
            pl.BlockSpec((C, C), lambda i: (0, 0)),
            pl.BlockSpec((1, C), lambda i: (0, 0)),
        ],
        out_specs=[
            pl.BlockSpec((n, C), lambda i: (0, 0)),
            pl.BlockSpec((n, CP), lambda i: (0, 0)),
        ],
        out_shape=[
            jax.ShapeDtypeStruct((n, C), jnp.float32),
            jax.ShapeDtypeStruct((n, CP), jnp.int32),
        ],
    )(x2, W1, b1, Wp, bp)


_MASK = 0xFFFF


def _sc_pool_body(g_hbm, idx_hbm, out_hbm, idx_v, rows, out_v, g_sh, sems):
    cid = lax.axis_index("c")
    sid = lax.axis_index("s")
    wid = sid * NC + cid
    base_node = wid * N_PER_W

    rows_per_tile = N_REAL // NS
    pltpu.sync_copy(g_hbm.at[pl.ds(sid * rows_per_tile, rows_per_tile)],
                    g_sh.at[pl.ds(sid * rows_per_tile, rows_per_tile)])
    pltpu.sync_copy(idx_hbm.at[pl.ds(base_node * K, N_PER_W * K)], idx_v)
    plsc.subcore_barrier()

    def issue(ci, b):
        pltpu.async_copy(
            g_sh.at[idx_v.at[pl.ds(ci * (CH * K), CH * K)]], rows[b],
            sems[b])

    def drain(b):
        pltpu.make_async_copy(
            g_sh.at[idx_v.at[pl.ds(0, CH * K)]], rows[b], sems[b]).wait()

    def compute(b, ci):
        rows_v = rows[b]

        def j_body(j, carry):
            row0 = j * K
            for c in range(CP // 16):
                acc_lo = None
                acc_hi = None
                for k0 in range(0, K, 8):
                    los, his = [], []
                    for k in range(k0, k0 + 8):
                        xv = rows_v[row0 + k, pl.ds(c * 16, 16)]
                        los.append(xv & _MASK)
                        his.append(lax.shift_right_logical(xv, 16))
                    while len(los) > 1:
                        los = [jnp.maximum(los[2 * i], los[2 * i + 1])
                               for i in range(len(los) // 2)]
                        his = [jnp.maximum(his[2 * i], his[2 * i + 1])
                               for i in range(len(his) // 2)]
                    if acc_lo is None:
                        acc_lo, acc_hi = los[0], his[0]
                    else:
                        acc_lo = jnp.maximum(acc_lo, los[0])
                        acc_hi = jnp.maximum(acc_hi, his[0])
                out_v[ci * CH + j, pl.ds(c * 16, 16)] = (
                    lax.shift_left(acc_hi, 16) | acc_lo)
            return carry

        lax.fori_loop(0, CH, j_body, 0)

    for b in range(NBUF - 1):
        issue(b, b)

    def group_body(gi, carry):
        c0 = gi * NBUF
        for b in range(NBUF):
            ci = c0 + b

            @pl.when(ci + NBUF - 1 < CHUNKS)
            def _():
                issue(ci + NBUF - 1, (b + NBUF - 1) % NBUF)

            drain(b)
            compute(b, ci)
        return carry

    lax.fori_loop(0, GROUPS, group_body, 0)
    pltpu.sync_copy(out_v, out_hbm.at[pl.ds(base_node, N_PER_W)])


_sc_pool = functools.partial(
    pl.kernel,
    out_type=jax.ShapeDtypeStruct((N_PAD, CP), jnp.int32),
    mesh=plsc.VectorSubcoreMesh(core_axis_name="c", subcore_axis_name="s"),
    scratch_types=[
        pltpu.VMEM((N_PER_W * K,), jnp.int32),
        [pltpu.VMEM((CH * K, CP), jnp.int32) for _ in range(NBUF)],
        pltpu.VMEM((N_PER_W, CP), jnp.int32),
        pltpu.VMEM_SHARED((N_PAD, CP), jnp.int32),
        [pltpu.SemaphoreType.DMA for _ in range(NBUF)],
    ],
    compiler_params=pltpu.CompilerParams(use_tc_tiling_on_sc=False),
)(_sc_pool_body)


def _stage3_body(h_ref, p_ref, w2a_ref, w2blo_ref, w2bhi_ref, b2_ref, o_ref):
    o = lax.dot_general(w2a_ref[...], h_ref[...], (((1,), (1,)), ((), ())),
                        preferred_element_type=jnp.float32)
    p = p_ref[...]
    p_lo = lax.bitcast_convert_type(lax.shift_left(p, 16), jnp.float32)
    p_hi = lax.bitcast_convert_type(p & (-65536), jnp.float32)
    o = o + lax.dot_general(w2blo_ref[...], p_lo, (((1,), (1,)), ((), ())),
                            preferred_element_type=jnp.float32)
    o = o + lax.dot_general(w2bhi_ref[...], p_hi, (((1,), (1,)), ((), ())),
                            preferred_element_type=jnp.float32)
    o_ref[...] = jnp.maximum(o + b2_ref[...], 0.0)


def _stage3(h, pooled, W2a, W2blo, W2bhi, b2, n):
    return pl.pallas_call(
        _stage3_body,
        grid=(1,),
        in_specs=[
            pl.BlockSpec((n, C), lambda i: (0, 0)),
            pl.BlockSpec((n, CP), lambda i: (0, 0)),
            pl.BlockSpec((C, C), lambda i: (0, 0)),
            pl.BlockSpec((C, CP), lambda i: (0, 0)),
            pl.BlockSpec((C, CP), lambda i: (0, 0)),
            pl.BlockSpec((C, 1), lambda i: (0, 0)),
        ],
        out_specs=pl.BlockSpec((C, n), lambda i: (0, 0)),
        out_shape=jax.ShapeDtypeStruct((C, n), jnp.float32),
    )(h, pooled, W2a, W2blo, W2bhi, b2)


def kernel(x, edge_index, W1, b1, Wp, bp, W2, b2):
    n = x.shape[2]
    x2 = x.reshape(C, n)
    idx = edge_index[0, 0]
    idx_flat = jnp.pad(idx, ((0, N_PAD - n), (0, 0))).reshape(-1)
    h, g_packed = _stage1(x2, W1, b1.reshape(1, C), Wp, bp.reshape(1, C), n)
    pooled_packed = _sc_pool(g_packed, idx_flat)
    out = _stage3(h, pooled_packed, W2[:, :C], W2[:, C:C + CP],
                  W2[:, C + CP:], b2.reshape(C, 1), n)
    return out.reshape(1, C, n, 1)
